# baseline (speedup 1.0000x reference)
import functools
import math

import jax
import jax.numpy as jnp
from jax import lax
from jax.experimental import pallas as pl
from jax.experimental.pallas import tpu as pltpu
from jax.experimental.pallas import tpu_sc as plsc

NI = 10000
NU = 10000
DD = 256
HH = 128
EE = 320000
ROWS = 10240
DUMMY = NI
NSUB = 16
SLAB = ROWS // NSUB
CH = 128
G = 16
C0 = 160
C1 = 80
BNC = 1.0 / math.sqrt(1.0 + 1e-5)
R = 400


def _zero_fill(buf):
    zeros = jnp.zeros((16,), jnp.float32)

    def row(r, carry):
        def col(k, c2):
            buf[r, pl.ds(k * 16, 16)] = zeros
            return c2
        return lax.fori_loop(0, HH // 16, col, carry)

    lax.fori_loop(0, CH, row, 0)


def _agg_run(tab, src_slab, dst_slab, sidx, didx, buf_a, buf_b, acc,
             sem_a, sem_b, ngroups):
    def group(g, carry):
        pltpu.sync_copy(src_slab.at[pl.ds(g * G, G)], sidx)
        pltpu.sync_copy(dst_slab.at[pl.ds(g * G, G)], didx)
        pltpu.async_copy(tab.at[sidx.at[0]], buf_a, sem_a)

        def pair(k, c2):
            c = 2 * k
            pltpu.make_async_copy(tab.at[sidx.at[c]], buf_a, sem_a).wait()
            pltpu.async_copy(tab.at[sidx.at[c + 1]], buf_b, sem_b)
            pltpu.sync_copy(buf_a, acc.at[didx.at[c]], add=True)
            pltpu.make_async_copy(tab.at[sidx.at[c + 1]], buf_b, sem_b).wait()

            @pl.when(k < G // 2 - 1)
            def _():
                pltpu.async_copy(tab.at[sidx.at[c + 2]], buf_a, sem_a)

            pltpu.sync_copy(buf_b, acc.at[didx.at[c + 1]], add=True)
            return c2

        return lax.fori_loop(0, G // 2, pair, carry)

    lax.fori_loop(0, ngroups, group, 0)


def _acc_init(buf, acc, sid):
    _zero_fill(buf)
    for j in range(SLAB // CH):
        pltpu.sync_copy(buf, acc.at[pl.ds(sid * SLAB + j * CH, CH)])
    plsc.subcore_barrier()


_SC_MESH = plsc.VectorSubcoreMesh(core_axis_name="c", subcore_axis_name="s")


@functools.partial(
    pl.kernel,
    mesh=_SC_MESH,
    out_type=[jax.ShapeDtypeStruct((ROWS, HH), jnp.float32),
              jax.ShapeDtypeStruct((ROWS, HH), jnp.float32)],
    scratch_types=[
        pltpu.VMEM((G, CH), jnp.int32),
        pltpu.VMEM((G, CH), jnp.int32),
        pltpu.VMEM((CH, HH), jnp.float32),
        pltpu.VMEM((CH, HH), jnp.float32),
        pltpu.VMEM_SHARED((ROWS, HH), jnp.float32),
        pltpu.SemaphoreType.DMA,
        pltpu.SemaphoreType.DMA,
    ],
)
def _agg_layer0(tab_u, tab_i, ub_src, ub_dst, iu_src, iu_dst,
                out_i, out_u, sidx, didx, buf_a, buf_b, acc, sem_a, sem_b):
    cid = lax.axis_index("c")
    sid = lax.axis_index("s")
    _acc_init(buf_a, acc, sid)

    @pl.when(cid == 0)
    def _():
        _agg_run(tab_u, ub_src.at[sid], ub_dst.at[sid], sidx, didx,
                 buf_a, buf_b, acc, sem_a, sem_b, C0 // G)

    @pl.when(cid != 0)
    def _():
        _agg_run(tab_i, iu_src.at[sid], iu_dst.at[sid], sidx, didx,
                 buf_a, buf_b, acc, sem_a, sem_b, C0 // G)

    plsc.subcore_barrier()

    @pl.when(cid == 0)
    def _():
        pltpu.sync_copy(acc.at[pl.ds(sid * SLAB, SLAB)],
                        out_i.at[pl.ds(sid * SLAB, SLAB)])

    @pl.when(cid != 0)
    def _():
        pltpu.sync_copy(acc.at[pl.ds(sid * SLAB, SLAB)],
                        out_u.at[pl.ds(sid * SLAB, SLAB)])


@functools.partial(
    pl.kernel,
    mesh=_SC_MESH,
    out_type=[jax.ShapeDtypeStruct((ROWS, HH), jnp.float32),
              jax.ShapeDtypeStruct((ROWS, HH), jnp.float32)],
    scratch_types=[
        pltpu.VMEM((G, CH), jnp.int32),
        pltpu.VMEM((G, CH), jnp.int32),
        pltpu.VMEM((CH, HH), jnp.float32),
        pltpu.VMEM((CH, HH), jnp.float32),
        pltpu.VMEM_SHARED((ROWS, HH), jnp.float32),
        pltpu.SemaphoreType.DMA,
        pltpu.SemaphoreType.DMA,
    ],
)
def _agg_layer1(tab, src4, dst4, out_a, out_b, sidx, didx, buf_a, buf_b, acc,
                sem_a, sem_b):
    cid = lax.axis_index("c")
    sid = lax.axis_index("s")
    _acc_init(buf_a, acc, sid)

    _agg_run(tab, src4.at[cid, sid], dst4.at[cid, sid], sidx, didx,
             buf_a, buf_b, acc, sem_a, sem_b, C1 // G)

    plsc.subcore_barrier()

    @pl.when(cid == 0)
    def _():
        pltpu.sync_copy(acc.at[pl.ds(sid * SLAB, SLAB)],
                        out_a.at[pl.ds(sid * SLAB, SLAB)])

    @pl.when(cid != 0)
    def _():
        pltpu.sync_copy(acc.at[pl.ds(sid * SLAB, SLAB)],
                        out_b.at[pl.ds(sid * SLAB, SLAB)])


def _pad_idx(idx, total):
    return jnp.concatenate(
        [idx.astype(jnp.int32),
         jnp.zeros((total - EE,), jnp.int32)])


def _pad_dst(idx, total):
    pad = jnp.arange(total - EE, dtype=jnp.int32) % (ROWS - NI) + DUMMY
    return jnp.concatenate([idx.astype(jnp.int32), pad])


def _stage_a_body(xi_ref, xu_ref, y_ref, emb_ref, wr_ub_ref, wr_iu_ref,
                  xi0_ref, pu0_ref, pi0_ref):
    y = y_ref[...]
    e0 = emb_ref[0:1, :]
    e1 = emb_ref[1:2, :]
    sel0 = jnp.where(y == 0, 1.0, 0.0)
    sel1 = jnp.where(y == 1, 1.0, 0.0)
    xi0 = xi_ref[...] + sel0 * e0 + sel1 * e1
    xi0_ref[...] = xi0
    pu0_ref[...] = jnp.dot(xu_ref[...], wr_ub_ref[...],
                           preferred_element_type=jnp.float32)
    pi0_ref[...] = jnp.dot(xi0, wr_iu_ref[...],
                           preferred_element_type=jnp.float32)


def _stage_b_body(aggi_ref, aggu_ref, xi0_ref, xu_ref, wro_ub_ref, wro_iu_ref,
                  brel_i_ref, brel_u_ref, g0i_ref, b0i_ref, g0u_ref, b0u_ref,
                  wr1_ref, xi1_ref, pu1_ref):
    ni = (aggi_ref[...] + brel_i_ref[...]
          + jnp.dot(xi0_ref[...], wro_ub_ref[...],
                    preferred_element_type=jnp.float32))
    xi1 = jnp.maximum(g0i_ref[...] * (ni * BNC) + b0i_ref[...], 0.0)
    nu = (aggu_ref[...] + brel_u_ref[...]
          + jnp.dot(xu_ref[...], wro_iu_ref[...],
                    preferred_element_type=jnp.float32))
    xu1 = jnp.maximum(g0u_ref[...] * (nu * BNC) + b0u_ref[...], 0.0)
    xi1_ref[...] = xi1
    pu1_ref[...] = jnp.dot(xu1, wr1_ref[...],
                           preferred_element_type=jnp.float32)


def _stage_c_body(a0_ref, a1_ref, xi1_ref, xi0_ref, wro1_ref, brel1_ref,
                  g1_ref, b1_ref, l1a_ref, l1b_ref, l1c_ref, l1bias_ref,
                  gl_ref, bl_ref, l2_ref, l2b_ref, out_ref):
    ni = (a0_ref[...] + a1_ref[...] + brel1_ref[...]
          + jnp.dot(xi1_ref[...], wro1_ref[...],
                    preferred_element_type=jnp.float32))
    xi2 = jnp.maximum(g1_ref[...] * (ni * BNC) + b1_ref[...], 0.0)
    h = (jnp.dot(xi0_ref[...], l1a_ref[...],
                 preferred_element_type=jnp.float32)
         + jnp.dot(xi1_ref[...], l1b_ref[...],
                   preferred_element_type=jnp.float32)
         + jnp.dot(xi2, l1c_ref[...], preferred_element_type=jnp.float32)
         + l1bias_ref[...])
    h = jnp.maximum(gl_ref[...] * (h * BNC) + bl_ref[...], 0.0)
    out_ref[...] = jnp.dot(h, l2_ref[...],
                           preferred_element_type=jnp.float32) + l2b_ref[...]


def _row_spec(d):
    return pl.BlockSpec((R, d), lambda i: (i, 0))


def _full_spec(a, b):
    return pl.BlockSpec((a, b), lambda i: (0, 0))


def kernel(x_item, x_user, edge_index_ub, edge_index_iu, y_emb, emb,
           W_rel0_ub, b_rel0_ub, W_root0_ub, W_rel0_iu, b_rel0_iu, W_root0_iu,
           bn0_item_g, bn0_item_b, bn0_user_g, bn0_user_b,
           W_rel1_ub, b_rel1_ub, W_root1_ub, W_rel1_iu, b_rel1_iu, W_root1_iu,
           bn1_item_g, bn1_item_b, bn1_user_g, bn1_user_b,
           lin1_W, lin1_b, bnl_g, bnl_b, lin2_W, lin2_b):
    f32 = jnp.float32
    row = lambda v: v.reshape(1, -1).astype(f32)

    xi0, pu0, pi0 = pl.pallas_call(
        _stage_a_body,
        grid=(NI // R,),
        in_specs=[_row_spec(DD), _row_spec(DD), _row_spec(1),
                  _full_spec(3, DD), _full_spec(DD, HH), _full_spec(DD, HH)],
        out_specs=[_row_spec(DD), _row_spec(HH), _row_spec(HH)],
        out_shape=[jax.ShapeDtypeStruct((NI, DD), f32),
                   jax.ShapeDtypeStruct((NU, HH), f32),
                   jax.ShapeDtypeStruct((NI, HH), f32)],
    )(x_item, x_user, y_emb.reshape(NI, 1).astype(jnp.int32),
      emb.astype(f32), W_rel0_ub, W_rel0_iu)

    tot0 = NSUB * C0 * CH
    ub_src = _pad_idx(edge_index_ub[0], tot0).reshape(NSUB, C0, CH)
    ub_dst = _pad_dst(edge_index_ub[1], tot0).reshape(NSUB, C0, CH)
    iu_src = _pad_idx(edge_index_iu[0], tot0).reshape(NSUB, C0, CH)
    iu_dst = _pad_dst(edge_index_iu[1], tot0).reshape(NSUB, C0, CH)
    aggi0, aggu0 = _agg_layer0(pu0, pi0, ub_src, ub_dst, iu_src, iu_dst)

    xi1, pu1 = pl.pallas_call(
        _stage_b_body,
        grid=(NI // R,),
        in_specs=[_row_spec(HH), _row_spec(HH), _row_spec(DD), _row_spec(DD),
                  _full_spec(DD, HH), _full_spec(DD, HH),
                  _full_spec(1, HH), _full_spec(1, HH), _full_spec(1, HH),
                  _full_spec(1, HH), _full_spec(1, HH), _full_spec(1, HH),
                  _full_spec(HH, HH)],
        out_specs=[_row_spec(HH), _row_spec(HH)],
        out_shape=[jax.ShapeDtypeStruct((NI, HH), f32),
                   jax.ShapeDtypeStruct((NU, HH), f32)],
    )(aggi0, aggu0, xi0, x_user, W_root0_ub, W_root0_iu,
      row(b_rel0_ub), row(b_rel0_iu), row(bn0_item_g), row(bn0_item_b),
      row(bn0_user_g), row(bn0_user_b), W_rel1_ub)

    tot1 = 2 * NSUB * C1 * CH
    src4 = _pad_idx(edge_index_ub[0], tot1).reshape(2, NSUB, C1, CH)
    dst4 = _pad_dst(edge_index_ub[1], tot1).reshape(2, NSUB, C1, CH)
    agg1a, agg1b = _agg_layer1(pu1, src4, dst4)

    l2p = jnp.pad(lin2_W, ((0, 0), (0, 6)))
    l2bp = jnp.pad(lin2_b, (0, 6))
    out8 = pl.pallas_call(
        _stage_c_body,
        grid=(NI // R,),
        in_specs=[_row_spec(HH), _row_spec(HH), _row_spec(HH), _row_spec(DD),
                  _full_spec(HH, HH), _full_spec(1, HH),
                  _full_spec(1, HH), _full_spec(1, HH),
                  _full_spec(DD, HH), _full_spec(HH, HH), _full_spec(HH, HH),
                  _full_spec(1, HH), _full_spec(1, HH), _full_spec(1, HH),
                  _full_spec(HH, 8), _full_spec(1, 8)],
        out_specs=[_row_spec(8)],
        out_shape=[jax.ShapeDtypeStruct((NI, 8), f32)],
    )(agg1a, agg1b, xi1, xi0, W_root1_ub, row(b_rel1_ub),
      row(bn1_item_g), row(bn1_item_b),
      lin1_W[:DD], lin1_W[DD:DD + HH], lin1_W[DD + HH:], row(lin1_b),
      row(bnl_g), row(bnl_b), l2p, row(l2bp))[0]

    return out8[:, :2]


# speedup vs baseline: 1.0115x; 1.0115x over previous
import functools
import math

import jax
import jax.numpy as jnp
from jax import lax
from jax.experimental import pallas as pl
from jax.experimental.pallas import tpu as pltpu
from jax.experimental.pallas import tpu_sc as plsc

NI = 10000
NU = 10000
DD = 256
HH = 128
EE = 320000
ROWS = 10240
DUMMY = NI
NSUB = 16
SLAB = ROWS // NSUB
CH = 128
G = 16
C0 = 160
C1 = 80
BNC = 1.0 / math.sqrt(1.0 + 1e-5)
R = 400


def _zero_fill(buf):
    zeros = jnp.zeros((16,), jnp.float32)

    def row(r, carry):
        def col(k, c2):
            buf[r, pl.ds(k * 16, 16)] = zeros
            return c2
        return lax.fori_loop(0, HH // 16, col, carry)

    lax.fori_loop(0, CH, row, 0)


def _agg_run(tab, src_slab, dst_slab, sidx, didx, buf_a, buf_b, acc,
             sem_a, sem_b, ngroups):
    def group(g, carry):
        pltpu.sync_copy(src_slab.at[pl.ds(g * G, G)], sidx)
        pltpu.sync_copy(dst_slab.at[pl.ds(g * G, G)], didx)
        pltpu.async_copy(tab.at[sidx.at[0]], buf_a, sem_a)

        def pair(k, c2):
            c = 2 * k
            pltpu.make_async_copy(tab.at[sidx.at[c]], buf_a, sem_a).wait()
            pltpu.async_copy(tab.at[sidx.at[c + 1]], buf_b, sem_b)
            pltpu.make_async_copy(tab.at[sidx.at[c + 1]], buf_b, sem_b).wait()

            @pl.when(k < G // 2 - 1)
            def _():
                pltpu.async_copy(tab.at[sidx.at[c + 2]], buf_a, sem_a)

            return c2

        return lax.fori_loop(0, G // 2, pair, carry)

    lax.fori_loop(0, ngroups, group, 0)


def _acc_init(buf, acc, sid):
    _zero_fill(buf)
    for j in range(SLAB // CH):
        pltpu.sync_copy(buf, acc.at[pl.ds(sid * SLAB + j * CH, CH)])
    plsc.subcore_barrier()


_SC_MESH = plsc.VectorSubcoreMesh(core_axis_name="c", subcore_axis_name="s")


@functools.partial(
    pl.kernel,
    mesh=_SC_MESH,
    out_type=[jax.ShapeDtypeStruct((ROWS, HH), jnp.float32),
              jax.ShapeDtypeStruct((ROWS, HH), jnp.float32)],
    scratch_types=[
        pltpu.VMEM((G, CH), jnp.int32),
        pltpu.VMEM((G, CH), jnp.int32),
        pltpu.VMEM((CH, HH), jnp.float32),
        pltpu.VMEM((CH, HH), jnp.float32),
        pltpu.VMEM_SHARED((ROWS, HH), jnp.float32),
        pltpu.SemaphoreType.DMA,
        pltpu.SemaphoreType.DMA,
    ],
)
def _agg_layer0(tab_u, tab_i, ub_src, ub_dst, iu_src, iu_dst,
                out_i, out_u, sidx, didx, buf_a, buf_b, acc, sem_a, sem_b):
    cid = lax.axis_index("c")
    sid = lax.axis_index("s")
    _acc_init(buf_a, acc, sid)

    @pl.when(cid == 0)
    def _():
        _agg_run(tab_u, ub_src.at[sid], ub_dst.at[sid], sidx, didx,
                 buf_a, buf_b, acc, sem_a, sem_b, C0 // G)

    @pl.when(cid != 0)
    def _():
        _agg_run(tab_i, iu_src.at[sid], iu_dst.at[sid], sidx, didx,
                 buf_a, buf_b, acc, sem_a, sem_b, C0 // G)

    plsc.subcore_barrier()

    @pl.when(cid == 0)
    def _():
        pltpu.sync_copy(acc.at[pl.ds(sid * SLAB, SLAB)],
                        out_i.at[pl.ds(sid * SLAB, SLAB)])

    @pl.when(cid != 0)
    def _():
        pltpu.sync_copy(acc.at[pl.ds(sid * SLAB, SLAB)],
                        out_u.at[pl.ds(sid * SLAB, SLAB)])


@functools.partial(
    pl.kernel,
    mesh=_SC_MESH,
    out_type=[jax.ShapeDtypeStruct((ROWS, HH), jnp.float32),
              jax.ShapeDtypeStruct((ROWS, HH), jnp.float32)],
    scratch_types=[
        pltpu.VMEM((G, CH), jnp.int32),
        pltpu.VMEM((G, CH), jnp.int32),
        pltpu.VMEM((CH, HH), jnp.float32),
        pltpu.VMEM((CH, HH), jnp.float32),
        pltpu.VMEM_SHARED((ROWS, HH), jnp.float32),
        pltpu.SemaphoreType.DMA,
        pltpu.SemaphoreType.DMA,
    ],
)
def _agg_layer1(tab, src4, dst4, out_a, out_b, sidx, didx, buf_a, buf_b, acc,
                sem_a, sem_b):
    cid = lax.axis_index("c")
    sid = lax.axis_index("s")
    _acc_init(buf_a, acc, sid)

    _agg_run(tab, src4.at[cid, sid], dst4.at[cid, sid], sidx, didx,
             buf_a, buf_b, acc, sem_a, sem_b, C1 // G)

    plsc.subcore_barrier()

    @pl.when(cid == 0)
    def _():
        pltpu.sync_copy(acc.at[pl.ds(sid * SLAB, SLAB)],
                        out_a.at[pl.ds(sid * SLAB, SLAB)])

    @pl.when(cid != 0)
    def _():
        pltpu.sync_copy(acc.at[pl.ds(sid * SLAB, SLAB)],
                        out_b.at[pl.ds(sid * SLAB, SLAB)])


def _pad_idx(idx, total):
    return jnp.concatenate(
        [idx.astype(jnp.int32),
         jnp.zeros((total - EE,), jnp.int32)])


def _pad_dst(idx, total):
    pad = jnp.arange(total - EE, dtype=jnp.int32) % (ROWS - NI) + DUMMY
    return jnp.concatenate([idx.astype(jnp.int32), pad])


def _stage_a_body(xi_ref, xu_ref, y_ref, emb_ref, wr_ub_ref, wr_iu_ref,
                  xi0_ref, pu0_ref, pi0_ref):
    y = y_ref[...]
    e0 = emb_ref[0:1, :]
    e1 = emb_ref[1:2, :]
    sel0 = jnp.where(y == 0, 1.0, 0.0)
    sel1 = jnp.where(y == 1, 1.0, 0.0)
    xi0 = xi_ref[...] + sel0 * e0 + sel1 * e1
    xi0_ref[...] = xi0
    pu0_ref[...] = jnp.dot(xu_ref[...], wr_ub_ref[...],
                           preferred_element_type=jnp.float32)
    pi0_ref[...] = jnp.dot(xi0, wr_iu_ref[...],
                           preferred_element_type=jnp.float32)


def _stage_b_body(aggi_ref, aggu_ref, xi0_ref, xu_ref, wro_ub_ref, wro_iu_ref,
                  brel_i_ref, brel_u_ref, g0i_ref, b0i_ref, g0u_ref, b0u_ref,
                  wr1_ref, xi1_ref, pu1_ref):
    ni = (aggi_ref[...] + brel_i_ref[...]
          + jnp.dot(xi0_ref[...], wro_ub_ref[...],
                    preferred_element_type=jnp.float32))
    xi1 = jnp.maximum(g0i_ref[...] * (ni * BNC) + b0i_ref[...], 0.0)
    nu = (aggu_ref[...] + brel_u_ref[...]
          + jnp.dot(xu_ref[...], wro_iu_ref[...],
                    preferred_element_type=jnp.float32))
    xu1 = jnp.maximum(g0u_ref[...] * (nu * BNC) + b0u_ref[...], 0.0)
    xi1_ref[...] = xi1
    pu1_ref[...] = jnp.dot(xu1, wr1_ref[...],
                           preferred_element_type=jnp.float32)


def _stage_c_body(a0_ref, a1_ref, xi1_ref, xi0_ref, wro1_ref, brel1_ref,
                  g1_ref, b1_ref, l1a_ref, l1b_ref, l1c_ref, l1bias_ref,
                  gl_ref, bl_ref, l2_ref, l2b_ref, out_ref):
    ni = (a0_ref[...] + a1_ref[...] + brel1_ref[...]
          + jnp.dot(xi1_ref[...], wro1_ref[...],
                    preferred_element_type=jnp.float32))
    xi2 = jnp.maximum(g1_ref[...] * (ni * BNC) + b1_ref[...], 0.0)
    h = (jnp.dot(xi0_ref[...], l1a_ref[...],
                 preferred_element_type=jnp.float32)
         + jnp.dot(xi1_ref[...], l1b_ref[...],
                   preferred_element_type=jnp.float32)
         + jnp.dot(xi2, l1c_ref[...], preferred_element_type=jnp.float32)
         + l1bias_ref[...])
    h = jnp.maximum(gl_ref[...] * (h * BNC) + bl_ref[...], 0.0)
    out_ref[...] = jnp.dot(h, l2_ref[...],
                           preferred_element_type=jnp.float32) + l2b_ref[...]


def _row_spec(d):
    return pl.BlockSpec((R, d), lambda i: (i, 0))


def _full_spec(a, b):
    return pl.BlockSpec((a, b), lambda i: (0, 0))


def kernel(x_item, x_user, edge_index_ub, edge_index_iu, y_emb, emb,
           W_rel0_ub, b_rel0_ub, W_root0_ub, W_rel0_iu, b_rel0_iu, W_root0_iu,
           bn0_item_g, bn0_item_b, bn0_user_g, bn0_user_b,
           W_rel1_ub, b_rel1_ub, W_root1_ub, W_rel1_iu, b_rel1_iu, W_root1_iu,
           bn1_item_g, bn1_item_b, bn1_user_g, bn1_user_b,
           lin1_W, lin1_b, bnl_g, bnl_b, lin2_W, lin2_b):
    f32 = jnp.float32
    row = lambda v: v.reshape(1, -1).astype(f32)

    xi0, pu0, pi0 = pl.pallas_call(
        _stage_a_body,
        grid=(NI // R,),
        in_specs=[_row_spec(DD), _row_spec(DD), _row_spec(1),
                  _full_spec(3, DD), _full_spec(DD, HH), _full_spec(DD, HH)],
        out_specs=[_row_spec(DD), _row_spec(HH), _row_spec(HH)],
        out_shape=[jax.ShapeDtypeStruct((NI, DD), f32),
                   jax.ShapeDtypeStruct((NU, HH), f32),
                   jax.ShapeDtypeStruct((NI, HH), f32)],
    )(x_item, x_user, y_emb.reshape(NI, 1).astype(jnp.int32),
      emb.astype(f32), W_rel0_ub, W_rel0_iu)

    tot0 = NSUB * C0 * CH
    ub_src = _pad_idx(edge_index_ub[0], tot0).reshape(NSUB, C0, CH)
    ub_dst = _pad_dst(edge_index_ub[1], tot0).reshape(NSUB, C0, CH)
    iu_src = _pad_idx(edge_index_iu[0], tot0).reshape(NSUB, C0, CH)
    iu_dst = _pad_dst(edge_index_iu[1], tot0).reshape(NSUB, C0, CH)
    aggi0, aggu0 = _agg_layer0(pu0, pi0, ub_src, ub_dst, iu_src, iu_dst)

    xi1, pu1 = pl.pallas_call(
        _stage_b_body,
        grid=(NI // R,),
        in_specs=[_row_spec(HH), _row_spec(HH), _row_spec(DD), _row_spec(DD),
                  _full_spec(DD, HH), _full_spec(DD, HH),
                  _full_spec(1, HH), _full_spec(1, HH), _full_spec(1, HH),
                  _full_spec(1, HH), _full_spec(1, HH), _full_spec(1, HH),
                  _full_spec(HH, HH)],
        out_specs=[_row_spec(HH), _row_spec(HH)],
        out_shape=[jax.ShapeDtypeStruct((NI, HH), f32),
                   jax.ShapeDtypeStruct((NU, HH), f32)],
    )(aggi0, aggu0, xi0, x_user, W_root0_ub, W_root0_iu,
      row(b_rel0_ub), row(b_rel0_iu), row(bn0_item_g), row(bn0_item_b),
      row(bn0_user_g), row(bn0_user_b), W_rel1_ub)

    tot1 = 2 * NSUB * C1 * CH
    src4 = _pad_idx(edge_index_ub[0], tot1).reshape(2, NSUB, C1, CH)
    dst4 = _pad_dst(edge_index_ub[1], tot1).reshape(2, NSUB, C1, CH)
    agg1a, agg1b = _agg_layer1(pu1, src4, dst4)

    l2p = jnp.pad(lin2_W, ((0, 0), (0, 6)))
    l2bp = jnp.pad(lin2_b, (0, 6))
    out8 = pl.pallas_call(
        _stage_c_body,
        grid=(NI // R,),
        in_specs=[_row_spec(HH), _row_spec(HH), _row_spec(HH), _row_spec(DD),
                  _full_spec(HH, HH), _full_spec(1, HH),
                  _full_spec(1, HH), _full_spec(1, HH),
                  _full_spec(DD, HH), _full_spec(HH, HH), _full_spec(HH, HH),
                  _full_spec(1, HH), _full_spec(1, HH), _full_spec(1, HH),
                  _full_spec(HH, 8), _full_spec(1, 8)],
        out_specs=[_row_spec(8)],
        out_shape=[jax.ShapeDtypeStruct((NI, 8), f32)],
    )(agg1a, agg1b, xi1, xi0, W_root1_ub, row(b_rel1_ub),
      row(bn1_item_g), row(bn1_item_b),
      lin1_W[:DD], lin1_W[DD:DD + HH], lin1_W[DD + HH:], row(lin1_b),
      row(bnl_g), row(bnl_b), l2p, row(l2bp))[0]

    return out8[:, :2]


# speedup vs baseline: 2.3390x; 2.3125x over previous
import functools
import math

import jax
import jax.numpy as jnp
from jax import lax
from jax.experimental import pallas as pl
from jax.experimental.pallas import tpu as pltpu
from jax.experimental.pallas import tpu_sc as plsc

NI = 10000
NU = 10000
DD = 256
HH = 128
EE = 320000
ROWS = 10240
DUMMY = NI
NSUB = 16
SLAB = ROWS // NSUB
CH = 128
G = 16
C0 = 160
C1 = 80
BNC = 1.0 / math.sqrt(1.0 + 1e-5)
R = 400


def _zero_fill(buf):
    zeros = jnp.zeros((16,), jnp.float32)

    def row(r, carry):
        def col(k, c2):
            buf[r, pl.ds(k * 16, 16)] = zeros
            return c2
        return lax.fori_loop(0, HH // 16, col, carry)

    lax.fori_loop(0, CH, row, 0)


def _agg_run(tab, src_slab, dst_slab, sidx, didx, buf_a, buf_b, acc,
             sem_a, sem_b, ngroups):
    def group(g, carry):
        pltpu.sync_copy(src_slab.at[pl.ds(g * G, G)], sidx)
        pltpu.sync_copy(dst_slab.at[pl.ds(g * G, G)], didx)
        pltpu.async_copy(tab.at[sidx.at[0]], buf_a, sem_a)

        def pair(k, c2):
            c = 2 * k
            pltpu.make_async_copy(tab.at[sidx.at[c]], buf_a, sem_a).wait()
            pltpu.async_copy(tab.at[sidx.at[c + 1]], buf_b, sem_b)
            pltpu.sync_copy(buf_a, acc.at[didx.at[c]], add=True)
            pltpu.make_async_copy(tab.at[sidx.at[c + 1]], buf_b, sem_b).wait()

            @pl.when(k < G // 2 - 1)
            def _():
                pltpu.async_copy(tab.at[sidx.at[c + 2]], buf_a, sem_a)

            pltpu.sync_copy(buf_b, acc.at[didx.at[c + 1]], add=True)
            return c2

        return lax.fori_loop(0, G // 2, pair, carry)

    lax.fori_loop(0, ngroups, group, 0)


def _acc_init(buf, acc, sid):
    _zero_fill(buf)
    for j in range(SLAB // CH):
        pltpu.sync_copy(buf, acc.at[pl.ds(sid * SLAB + j * CH, CH)])
    plsc.subcore_barrier()


_SC_MESH = plsc.VectorSubcoreMesh(core_axis_name="c", subcore_axis_name="s")


@functools.partial(
    pl.kernel,
    mesh=_SC_MESH,
    out_type=[jax.ShapeDtypeStruct((ROWS, HH), jnp.float32),
              jax.ShapeDtypeStruct((ROWS, HH), jnp.float32)],
    scratch_types=[
        pltpu.VMEM((G, CH), jnp.int32),
        pltpu.VMEM((G, CH), jnp.int32),
        pltpu.VMEM((CH, HH), jnp.float32),
        pltpu.VMEM((CH, HH), jnp.float32),
        pltpu.VMEM_SHARED((ROWS, HH), jnp.float32),
        pltpu.SemaphoreType.DMA,
        pltpu.SemaphoreType.DMA,
    ],
)
def _agg_layer0(tab_u, tab_i, ub_src, ub_dst, iu_src, iu_dst,
                out_i, out_u, sidx, didx, buf_a, buf_b, acc, sem_a, sem_b):
    cid = lax.axis_index("c")
    sid = lax.axis_index("s")
    _acc_init(buf_a, acc, sid)

    @pl.when(cid == 0)
    def _():
        _agg_run(tab_u, ub_src.at[sid], ub_dst.at[sid], sidx, didx,
                 buf_a, buf_b, acc, sem_a, sem_b, C0 // G)

    @pl.when(cid != 0)
    def _():
        _agg_run(tab_i, iu_src.at[sid], iu_dst.at[sid], sidx, didx,
                 buf_a, buf_b, acc, sem_a, sem_b, C0 // G)

    plsc.subcore_barrier()

    @pl.when(cid == 0)
    def _():
        pltpu.sync_copy(acc.at[pl.ds(sid * SLAB, SLAB)],
                        out_i.at[pl.ds(sid * SLAB, SLAB)])

    @pl.when(cid != 0)
    def _():
        pltpu.sync_copy(acc.at[pl.ds(sid * SLAB, SLAB)],
                        out_u.at[pl.ds(sid * SLAB, SLAB)])


@functools.partial(
    pl.kernel,
    mesh=_SC_MESH,
    out_type=[jax.ShapeDtypeStruct((ROWS, HH), jnp.float32),
              jax.ShapeDtypeStruct((ROWS, HH), jnp.float32)],
    scratch_types=[
        pltpu.VMEM((G, CH), jnp.int32),
        pltpu.VMEM((G, CH), jnp.int32),
        pltpu.VMEM((CH, HH), jnp.float32),
        pltpu.VMEM((CH, HH), jnp.float32),
        pltpu.VMEM_SHARED((ROWS, HH), jnp.float32),
        pltpu.SemaphoreType.DMA,
        pltpu.SemaphoreType.DMA,
    ],
)
def _agg_layer1(tab, src4, dst4, out_a, out_b, sidx, didx, buf_a, buf_b, acc,
                sem_a, sem_b):
    cid = lax.axis_index("c")
    sid = lax.axis_index("s")
    _acc_init(buf_a, acc, sid)

    _agg_run(tab, src4.at[cid, sid], dst4.at[cid, sid], sidx, didx,
             buf_a, buf_b, acc, sem_a, sem_b, C1 // G)

    plsc.subcore_barrier()

    @pl.when(cid == 0)
    def _():
        pltpu.sync_copy(acc.at[pl.ds(sid * SLAB, SLAB)],
                        out_a.at[pl.ds(sid * SLAB, SLAB)])

    @pl.when(cid != 0)
    def _():
        pltpu.sync_copy(acc.at[pl.ds(sid * SLAB, SLAB)],
                        out_b.at[pl.ds(sid * SLAB, SLAB)])


def _pad_idx(idx, total):
    pad = jnp.arange(total - EE, dtype=jnp.int32) % NI
    return jnp.concatenate([idx.astype(jnp.int32), pad])


def _pad_dst(idx, total):
    pad = jnp.arange(total - EE, dtype=jnp.int32) % (ROWS - NI) + DUMMY
    return jnp.concatenate([idx.astype(jnp.int32), pad])


def _stage_a_body(xi_ref, xu_ref, y_ref, emb_ref, wr_ub_ref, wr_iu_ref,
                  xi0_ref, pu0_ref, pi0_ref):
    y = y_ref[...]
    e0 = emb_ref[0:1, :]
    e1 = emb_ref[1:2, :]
    sel0 = jnp.where(y == 0, 1.0, 0.0)
    sel1 = jnp.where(y == 1, 1.0, 0.0)
    xi0 = xi_ref[...] + sel0 * e0 + sel1 * e1
    xi0_ref[...] = xi0
    pu0_ref[...] = jnp.dot(xu_ref[...], wr_ub_ref[...],
                           preferred_element_type=jnp.float32)
    pi0_ref[...] = jnp.dot(xi0, wr_iu_ref[...],
                           preferred_element_type=jnp.float32)


def _stage_b_body(aggi_ref, aggu_ref, xi0_ref, xu_ref, wro_ub_ref, wro_iu_ref,
                  brel_i_ref, brel_u_ref, g0i_ref, b0i_ref, g0u_ref, b0u_ref,
                  wr1_ref, xi1_ref, pu1_ref):
    ni = (aggi_ref[...] + brel_i_ref[...]
          + jnp.dot(xi0_ref[...], wro_ub_ref[...],
                    preferred_element_type=jnp.float32))
    xi1 = jnp.maximum(g0i_ref[...] * (ni * BNC) + b0i_ref[...], 0.0)
    nu = (aggu_ref[...] + brel_u_ref[...]
          + jnp.dot(xu_ref[...], wro_iu_ref[...],
                    preferred_element_type=jnp.float32))
    xu1 = jnp.maximum(g0u_ref[...] * (nu * BNC) + b0u_ref[...], 0.0)
    xi1_ref[...] = xi1
    pu1_ref[...] = jnp.dot(xu1, wr1_ref[...],
                           preferred_element_type=jnp.float32)


def _stage_c_body(a0_ref, a1_ref, xi1_ref, xi0_ref, wro1_ref, brel1_ref,
                  g1_ref, b1_ref, l1a_ref, l1b_ref, l1c_ref, l1bias_ref,
                  gl_ref, bl_ref, l2_ref, l2b_ref, out_ref):
    ni = (a0_ref[...] + a1_ref[...] + brel1_ref[...]
          + jnp.dot(xi1_ref[...], wro1_ref[...],
                    preferred_element_type=jnp.float32))
    xi2 = jnp.maximum(g1_ref[...] * (ni * BNC) + b1_ref[...], 0.0)
    h = (jnp.dot(xi0_ref[...], l1a_ref[...],
                 preferred_element_type=jnp.float32)
         + jnp.dot(xi1_ref[...], l1b_ref[...],
                   preferred_element_type=jnp.float32)
         + jnp.dot(xi2, l1c_ref[...], preferred_element_type=jnp.float32)
         + l1bias_ref[...])
    h = jnp.maximum(gl_ref[...] * (h * BNC) + bl_ref[...], 0.0)
    out_ref[...] = jnp.dot(h, l2_ref[...],
                           preferred_element_type=jnp.float32) + l2b_ref[...]


def _row_spec(d):
    return pl.BlockSpec((R, d), lambda i: (i, 0))


def _full_spec(a, b):
    return pl.BlockSpec((a, b), lambda i: (0, 0))


def kernel(x_item, x_user, edge_index_ub, edge_index_iu, y_emb, emb,
           W_rel0_ub, b_rel0_ub, W_root0_ub, W_rel0_iu, b_rel0_iu, W_root0_iu,
           bn0_item_g, bn0_item_b, bn0_user_g, bn0_user_b,
           W_rel1_ub, b_rel1_ub, W_root1_ub, W_rel1_iu, b_rel1_iu, W_root1_iu,
           bn1_item_g, bn1_item_b, bn1_user_g, bn1_user_b,
           lin1_W, lin1_b, bnl_g, bnl_b, lin2_W, lin2_b):
    f32 = jnp.float32
    row = lambda v: v.reshape(1, -1).astype(f32)

    xi0, pu0, pi0 = pl.pallas_call(
        _stage_a_body,
        grid=(NI // R,),
        in_specs=[_row_spec(DD), _row_spec(DD), _row_spec(1),
                  _full_spec(3, DD), _full_spec(DD, HH), _full_spec(DD, HH)],
        out_specs=[_row_spec(DD), _row_spec(HH), _row_spec(HH)],
        out_shape=[jax.ShapeDtypeStruct((NI, DD), f32),
                   jax.ShapeDtypeStruct((NU, HH), f32),
                   jax.ShapeDtypeStruct((NI, HH), f32)],
    )(x_item, x_user, y_emb.reshape(NI, 1).astype(jnp.int32),
      emb.astype(f32), W_rel0_ub, W_rel0_iu)

    tot0 = NSUB * C0 * CH
    ub_src = _pad_idx(edge_index_ub[0], tot0).reshape(NSUB, C0, CH)
    ub_dst = _pad_dst(edge_index_ub[1], tot0).reshape(NSUB, C0, CH)
    iu_src = _pad_idx(edge_index_iu[0], tot0).reshape(NSUB, C0, CH)
    iu_dst = _pad_dst(edge_index_iu[1], tot0).reshape(NSUB, C0, CH)
    aggi0, aggu0 = _agg_layer0(pu0, pi0, ub_src, ub_dst, iu_src, iu_dst)

    xi1, pu1 = pl.pallas_call(
        _stage_b_body,
        grid=(NI // R,),
        in_specs=[_row_spec(HH), _row_spec(HH), _row_spec(DD), _row_spec(DD),
                  _full_spec(DD, HH), _full_spec(DD, HH),
                  _full_spec(1, HH), _full_spec(1, HH), _full_spec(1, HH),
                  _full_spec(1, HH), _full_spec(1, HH), _full_spec(1, HH),
                  _full_spec(HH, HH)],
        out_specs=[_row_spec(HH), _row_spec(HH)],
        out_shape=[jax.ShapeDtypeStruct((NI, HH), f32),
                   jax.ShapeDtypeStruct((NU, HH), f32)],
    )(aggi0, aggu0, xi0, x_user, W_root0_ub, W_root0_iu,
      row(b_rel0_ub), row(b_rel0_iu), row(bn0_item_g), row(bn0_item_b),
      row(bn0_user_g), row(bn0_user_b), W_rel1_ub)

    tot1 = 2 * NSUB * C1 * CH
    src4 = _pad_idx(edge_index_ub[0], tot1).reshape(2, NSUB, C1, CH)
    dst4 = _pad_dst(edge_index_ub[1], tot1).reshape(2, NSUB, C1, CH)
    agg1a, agg1b = _agg_layer1(pu1, src4, dst4)

    l2p = jnp.pad(lin2_W, ((0, 0), (0, 6)))
    l2bp = jnp.pad(lin2_b, (0, 6))
    out8 = pl.pallas_call(
        _stage_c_body,
        grid=(NI // R,),
        in_specs=[_row_spec(HH), _row_spec(HH), _row_spec(HH), _row_spec(DD),
                  _full_spec(HH, HH), _full_spec(1, HH),
                  _full_spec(1, HH), _full_spec(1, HH),
                  _full_spec(DD, HH), _full_spec(HH, HH), _full_spec(HH, HH),
                  _full_spec(1, HH), _full_spec(1, HH), _full_spec(1, HH),
                  _full_spec(HH, 8), _full_spec(1, 8)],
        out_specs=[_row_spec(8)],
        out_shape=[jax.ShapeDtypeStruct((NI, 8), f32)],
    )(agg1a, agg1b, xi1, xi0, W_root1_ub, row(b_rel1_ub),
      row(bn1_item_g), row(bn1_item_b),
      lin1_W[:DD], lin1_W[DD:DD + HH], lin1_W[DD + HH:], row(lin1_b),
      row(bnl_g), row(bnl_b), l2p, row(l2bp))[0]

    return out8[:, :2]
